# baseline (speedup 1.0000x reference)
import jax
import jax.numpy as jnp
from jax import lax
from jax.experimental import pallas as pl
from jax.experimental.pallas import tpu as pltpu
from jax.experimental.pallas import tpu_sc as plsc

_B, _L, _H, _D = 8, 4096, 8, 128
_Q = 8
_NC, _NS = 2, 16
_WPB = (_NC * _NS) // _B
_TPW = _Q // _WPB


def _scatter_body(len_hbm, nl_hbm, nk_hbm, nv_hbm, k_ref, v_ref, ul_hbm,
                  len_v, nl_v, ul_v, sem_len, sem_row):
  c = lax.axis_index("c")
  s = lax.axis_index("s")
  wid = s * _NC + c

  pltpu.async_copy(len_hbm, len_v.at[pl.ds(0, _B)], sem_len)
  pltpu.async_copy(nl_hbm, nl_v.at[pl.ds(0, _B)], sem_len)
  pltpu.make_async_copy(len_hbm, len_v.at[pl.ds(0, _B)], sem_len).wait()
  pltpu.make_async_copy(nl_hbm, nl_v.at[pl.ds(0, _B)], sem_len).wait()
  vals_l = len_v[...]
  vals_nl = nl_v[...]

  def _for_owned_slots(fn):
    for b in range(_B):
      l_b = vals_l[b]
      nl_b = vals_nl[b]
      owned = wid // _WPB == b
      for t in range(_TPW):
        j = lax.rem(wid, _WPB) * _TPW + t
        pl.when(jnp.logical_and(owned, j < nl_b))(
            lambda b=b, j=j, l_b=l_b: fn(b, j, l_b))

  def _fire(b, j, l_b):
    pltpu.async_copy(nk_hbm.at[b, j], k_ref.at[b, l_b + j], sem_row)
    pltpu.async_copy(nv_hbm.at[b, j], v_ref.at[b, l_b + j], sem_row)

  def _drain(b, j, l_b):
    pltpu.make_async_copy(nk_hbm.at[b, j], k_ref.at[b, l_b + j],
                          sem_row).wait()
    pltpu.make_async_copy(nv_hbm.at[b, j], v_ref.at[b, l_b + j],
                          sem_row).wait()

  _for_owned_slots(_fire)
  _for_owned_slots(_drain)

  @pl.when(jnp.logical_and(c == 0, s == 0))
  def _update_lengths():
    ul_v[...] = vals_l + vals_nl
    pltpu.sync_copy(ul_v.at[pl.ds(0, _B)], ul_hbm)


_sc_scatter = pl.kernel(
    _scatter_body,
    out_type=jax.ShapeDtypeStruct((_B,), jnp.int32),
    mesh=plsc.VectorSubcoreMesh(
        core_axis_name="c", subcore_axis_name="s",
        num_cores=_NC, num_subcores=_NS),
    scratch_types=[
        pltpu.VMEM((16,), jnp.int32),
        pltpu.VMEM((16,), jnp.int32),
        pltpu.VMEM((16,), jnp.int32),
        pltpu.SemaphoreType.DMA,
        pltpu.SemaphoreType.DMA,
    ],
)


def kernel(keys, values, lengths, new_keys, new_values, new_lengths):
  k_ref = jax.new_ref(keys)
  v_ref = jax.new_ref(values)
  updated_lengths = _sc_scatter(
      lengths, new_lengths, new_keys, new_values, k_ref, v_ref)
  return jax.freeze(k_ref), jax.freeze(v_ref), updated_lengths


# speedup vs baseline: 1.0363x; 1.0363x over previous
import jax
import jax.numpy as jnp
from jax import lax
from jax.experimental import pallas as pl
from jax.experimental.pallas import tpu as pltpu
from jax.experimental.pallas import tpu_sc as plsc

_B, _L, _H, _D = 8, 4096, 8, 128
_Q = 8
_NC, _NS = 2, 16
_WPB = (_NC * _NS) // _B
_TPW = _Q // _WPB


def _scatter_body(len_hbm, nl_hbm, nk_hbm, nv_hbm, k_ref, v_ref, ul_hbm,
                  len_v, nl_v, ul_v, rowk, rowv, sem_len, sem_row):
  c = lax.axis_index("c")
  s = lax.axis_index("s")
  wid = s * _NC + c

  pltpu.async_copy(len_hbm, len_v.at[pl.ds(0, _B)], sem_len)
  pltpu.async_copy(nl_hbm, nl_v.at[pl.ds(0, _B)], sem_len)
  pltpu.make_async_copy(len_hbm, len_v.at[pl.ds(0, _B)], sem_len).wait()
  pltpu.make_async_copy(nl_hbm, nl_v.at[pl.ds(0, _B)], sem_len).wait()
  vals_l = len_v[...]
  vals_nl = nl_v[...]

  def _for_owned_slots(fn):
    for b in range(_B):
      l_b = vals_l[b]
      nl_b = vals_nl[b]
      owned = wid // _WPB == b
      for t in range(_TPW):
        j = lax.rem(wid, _WPB) * _TPW + t
        pl.when(jnp.logical_and(owned, j < nl_b))(
            lambda b=b, j=j, l_b=l_b: fn(b, j, l_b))

  def _copy_rows(b, j, l_b):
    hk = pltpu.async_copy(nk_hbm.at[b, j], rowk, sem_row)
    hv = pltpu.async_copy(nv_hbm.at[b, j], rowv, sem_row)
    hk.wait()
    hv.wait()
    wk = pltpu.async_copy(rowk, k_ref.at[b, l_b + j], sem_row)
    wv = pltpu.async_copy(rowv, v_ref.at[b, l_b + j], sem_row)
    wk.wait()
    wv.wait()

  _for_owned_slots(_copy_rows)

  @pl.when(jnp.logical_and(c == 0, s == 0))
  def _update_lengths():
    ul_v[...] = vals_l + vals_nl
    pltpu.sync_copy(ul_v.at[pl.ds(0, _B)], ul_hbm)


_sc_scatter = pl.kernel(
    _scatter_body,
    out_type=jax.ShapeDtypeStruct((_B,), jnp.int32),
    mesh=plsc.VectorSubcoreMesh(
        core_axis_name="c", subcore_axis_name="s",
        num_cores=_NC, num_subcores=_NS),
    scratch_types=[
        pltpu.VMEM((16,), jnp.int32),
        pltpu.VMEM((16,), jnp.int32),
        pltpu.VMEM((16,), jnp.int32),
        pltpu.VMEM((_H, _D), jnp.float32),
        pltpu.VMEM((_H, _D), jnp.float32),
        pltpu.SemaphoreType.DMA,
        pltpu.SemaphoreType.DMA,
    ],
)


def kernel(keys, values, lengths, new_keys, new_values, new_lengths):
  k_ref = jax.new_ref(keys)
  v_ref = jax.new_ref(values)
  updated_lengths = _sc_scatter(
      lengths, new_lengths, new_keys, new_values, k_ref, v_ref)
  return jax.freeze(k_ref), jax.freeze(v_ref), updated_lengths
